# Initial kernel scaffold; baseline (speedup 1.0000x reference)
import jax
import jax.numpy as jnp
from jax.experimental import pallas as pl


def kernel(input, emb_table, fc_w, fc_b):
    raise NotImplementedError("write your pallas kernel here")


# speedup vs baseline: 3.4858x; 3.4858x over previous
import functools

import jax
import jax.numpy as jnp
from jax import lax
from jax.experimental import pallas as pl
from jax.experimental.pallas import tpu as pltpu
from jax.experimental.pallas import tpu_sc as plsc

_V, _D, _SEQ, _C = 21129, 768, 40, 12
_B = 4096
_CP = 16
_NW = 32
_BPW = _B // _NW
_SBLK = 512


@functools.partial(
    pl.kernel,
    out_type=jax.ShapeDtypeStruct((_B, _D), jnp.float32),
    mesh=plsc.VectorSubcoreMesh(core_axis_name="c", subcore_axis_name="s"),
    compiler_params=pltpu.CompilerParams(use_tc_tiling_on_sc=False),
    scratch_types=[
        pltpu.VMEM((_SEQ, _BPW), jnp.int32),
        pltpu.VMEM((_BPW, _D), jnp.float32),
        pltpu.SemaphoreType.DMA,
    ],
)
def _sc_pool(emb_hbm, idx_hbm, out_hbm, idx_v, acc_v, sem):
    wid = lax.axis_index("s") * 2 + lax.axis_index("c")
    pltpu.sync_copy(idx_hbm.at[pl.ds(wid * _SEQ, _SEQ)], idx_v)
    pltpu.async_copy(emb_hbm.at[idx_v.at[0]], acc_v, sem).wait()
    for t in range(1, _SEQ):
        pltpu.async_copy(emb_hbm.at[idx_v.at[t]], acc_v, sem, add=True).wait()
    pltpu.sync_copy(acc_v, out_hbm.at[pl.ds(wid * _BPW, _BPW)])


def _tc_head_body(sum_ref, w_ref, b_ref, out_ref):
    m = sum_ref[...] / jnp.float32(_SEQ)
    logits = lax.dot_general(
        m.astype(jnp.bfloat16), w_ref[...].astype(jnp.bfloat16),
        (((1,), (1,)), ((), ())),
        preferred_element_type=jnp.float32) + b_ref[...]
    p = jnp.exp(logits - jnp.max(logits, axis=1, keepdims=True))
    q = p / jnp.sum(p, axis=1, keepdims=True)
    mx = jnp.max(q, axis=1, keepdims=True)
    lanec = lax.broadcasted_iota(jnp.int32, q.shape, 1)
    cand = jnp.where(q == mx, lanec, _CP)
    out_ref[...] = jnp.min(cand, axis=1)


def _tc_head(sums, w_pad, b_pad):
    return pl.pallas_call(
        _tc_head_body,
        grid=(_B // _SBLK,),
        in_specs=[
            pl.BlockSpec((_SBLK, _D), lambda i: (i, 0)),
            pl.BlockSpec((_CP, _D), lambda i: (0, 0)),
            pl.BlockSpec((1, _CP), lambda i: (0, 0)),
        ],
        out_specs=pl.BlockSpec((_SBLK,), lambda i: (i,)),
        out_shape=jax.ShapeDtypeStruct((_B,), jnp.int32),
    )(sums, w_pad, b_pad)


def kernel(input, emb_table, fc_w, fc_b):
    idx = (input.reshape(_NW, _BPW, _SEQ)
           .transpose(0, 2, 1)
           .reshape(_NW * _SEQ, _BPW))
    sums = _sc_pool(emb_table, idx)
    w_pad = jnp.zeros((_CP, _D), jnp.float32).at[:_C].set(fc_w)
    b_pad = jnp.full((1, _CP), -1e30, jnp.float32).at[0, :_C].set(fc_b)
    return _tc_head(sums, w_pad, b_pad)
